# baseline (speedup 1.0000x reference)
import jax
import jax.numpy as jnp
from jax import lax
from jax.experimental import pallas as pl
from jax.experimental.pallas import tpu as pltpu
from jax.experimental.pallas import tpu_sc as plsc

NC, NS = 2, 16
NW = NC * NS
V = 1000000
D = 64
NB = 7812
PACKED = V // 2
BPW = 245

BATCH = 4096
HIST = 50
TOTAL = BATCH * HIST
PER_W = TOTAL // NW
CHUNK = 128
K = PER_W // CHUNK
OUTP = TOTAL // 2


BLK = 512
TCGRID = (V + BLK - 1) // BLK
PACKED_PAD = TCGRID * (BLK // 2)


def _tc_body(tabT_ref, out_ref):
    blk = tabT_ref[...]
    t1 = jnp.transpose(blk, (1, 0))
    out_ref[...] = jnp.concatenate([t1[:BLK // 2], t1[BLK // 2:]], axis=1)


def _tc_relayout(tabT):
    return pl.pallas_call(
        _tc_body,
        out_shape=jax.ShapeDtypeStruct((PACKED_PAD, 128), jnp.float32),
        grid=(TCGRID,),
        in_specs=[pl.BlockSpec((D, BLK), lambda i: (0, i))],
        out_specs=pl.BlockSpec((BLK // 2, 128), lambda i: (i, 0)),
    )(tabT)


def _gather_body(ids_hbm, packed_hbm, out_hbm, idx_v, pidx_v,
                 gb0, gb1, gb2, ob0, ob1, gs0, gs1, gs2, os0, os1):
    gb = (gb0, gb1, gb2)
    ob = (ob0, ob1)
    gsem = (gs0, gs1, gs2)
    osem = (os0, os1)
    wid = lax.axis_index("s") * NC + lax.axis_index("c")
    obase = wid * (PER_W // 2)
    pltpu.sync_copy(ids_hbm.at[wid], idx_v)

    def mk_pidx(j, g):
        for gg in range(8):
            v = idx_v[j, pl.ds(gg * 16, 16)]
            prow = lax.shift_left(lax.shift_right_logical(v, 9), 8) + (v & 255)
            pidx_v[g, pl.ds(gg * 16, 16)] = prow

    def gather_start(j, g):
        pltpu.async_copy(packed_hbm.at[pidx_v.at[g]], gb[g], gsem[g])

    def gather_wait(j, g):
        pltpu.make_async_copy(packed_hbm.at[pidx_v.at[g]], gb[g],
                              gsem[g]).wait()

    def extract(j, g, o):
        gbf = gb[g]
        obf = ob[o]
        half = (j % 2) * 64
        iota = lax.iota(jnp.int32, 16)

        def tgrp(t, _):
            rvec = iota + 16 * t
            idxv = idx_v[j, pl.ds(16 * t, 16)]
            parv = (lax.shift_right_logical(idxv, 8) & 1) * 64
            orowv = half + lax.shift_right_logical(rvec, 1)
            ocolv = (rvec & 1) * 64

            @plsc.parallel_loop(0, 64, unroll=4)
            def _(c):
                vals = plsc.load_gather(gbf, [rvec, parv + c])
                plsc.store_scatter(obf, [orowv, ocolv + c], vals)

            return _

        lax.fori_loop(0, 8, tgrp, None)

    def write_start(grp, o):
        pltpu.async_copy(ob[o], out_hbm.at[pl.ds(obase + grp * 128, 128)],
                         osem[o])

    def write_wait(grp, o):
        pltpu.make_async_copy(ob[o],
                              out_hbm.at[pl.ds(obase + grp * 128, 128)],
                              osem[o]).wait()

    mk_pidx(0, 0)
    gather_start(0, 0)
    mk_pidx(1, 1)
    gather_start(1, 1)
    for j in (0, 1, 2, 3):
        g = j % 3
        o = (j // 2) % 2
        if j + 2 < K:
            g2 = (j + 2) % 3
            mk_pidx(j + 2, g2)
            gather_start(j + 2, g2)
        gather_wait(j, g)
        extract(j, g, o)
        if j % 2 == 1:
            write_start(j // 2, o)

    def grp12(q, _):
        for s in range(12):
            j = 4 + 12 * q + s
            g = (4 + s) % 3
            o = ((4 + s) // 2) % 2
            g2 = (4 + s + 2) % 3
            mk_pidx(j + 2, g2)
            gather_start(j + 2, g2)
            gather_wait(j, g)
            if s % 2 == 0:
                write_wait((j - 4) // 2, o)
            extract(j, g, o)
            if s % 2 == 1:
                write_start(j // 2, o)
        return _

    lax.fori_loop(0, 3, grp12, None)

    for j in range(40, K):
        g = j % 3
        o = (j // 2) % 2
        if j + 2 < K:
            g2 = (j + 2) % 3
            mk_pidx(j + 2, g2)
            gather_start(j + 2, g2)
        gather_wait(j, g)
        if j % 2 == 0:
            write_wait((j - 4) // 2, o)
        extract(j, g, o)
        if j % 2 == 1:
            write_start(j // 2, o)

    write_wait(23, 1)
    write_wait(24, 0)


def _gather(ids3, packed):
    mesh = plsc.VectorSubcoreMesh(core_axis_name="c", subcore_axis_name="s")
    run = pl.kernel(
        _gather_body,
        out_type=jax.ShapeDtypeStruct((OUTP, 128), jnp.float32),
        mesh=mesh,
        scratch_types=[
            pltpu.VMEM((K, CHUNK), jnp.int32),
            pltpu.VMEM((3, CHUNK), jnp.int32),
            pltpu.VMEM((CHUNK, 128), jnp.float32),
            pltpu.VMEM((CHUNK, 128), jnp.float32),
            pltpu.VMEM((CHUNK, 128), jnp.float32),
            pltpu.VMEM((CHUNK, 128), jnp.float32),
            pltpu.VMEM((CHUNK, 128), jnp.float32),
            pltpu.SemaphoreType.DMA,
            pltpu.SemaphoreType.DMA,
            pltpu.SemaphoreType.DMA,
            pltpu.SemaphoreType.DMA,
            pltpu.SemaphoreType.DMA,
        ],
        compiler_params=pltpu.CompilerParams(use_tc_tiling_on_sc=True,
                                             needs_layout_passes=False,
                                             disable_bounds_checks=True,
                                             disable_semaphore_checks=True),
    )
    return run(ids3, packed)


@jax.jit
def _fused(input_ids, table):
    packed = _tc_relayout(table.T)
    ids3 = input_ids.astype(jnp.int32).reshape(NW, K, CHUNK)
    out2 = _gather(ids3, packed)
    return out2.reshape(BATCH, HIST, D)


def kernel(input_ids, embed_tokens_weight):
    return _fused(input_ids, embed_tokens_weight)


# speedup vs baseline: 1.9853x; 1.9853x over previous
import jax
import jax.numpy as jnp
from jax import lax
from jax.experimental import pallas as pl
from jax.experimental.pallas import tpu as pltpu
from jax.experimental.pallas import tpu_sc as plsc

NC = 2
NS = 16
NW = NC * NS

BATCH = 4096
HIST = 50
EMBED_DIM = 64

TOTAL = BATCH * HIST
PER_W = TOTAL // NW
CHUNK = 128
K = PER_W // CHUNK
PH = 5
ROWS = PH * CHUNK
NPH = K // PH


def _gather_body(ids_hbm, table_hbm, out_hbm, idx_v, big0, big1, gs0, gs1,
                 os0, os1):
    big = (big0, big1)
    gsem = (gs0, gs1)
    osem = (os0, os1)
    wid = lax.axis_index("s") * NC + lax.axis_index("c")
    base = wid * PER_W
    pltpu.sync_copy(ids_hbm.at[wid], idx_v)

    def fire(t, p):
        for c in range(PH):
            pltpu.async_copy(table_hbm.at[idx_v.at[t * PH + c]],
                             big[p].at[pl.ds(c * CHUNK, CHUNK)], gsem[p])

    def drain(t, p):
        for c in range(PH):
            pltpu.make_async_copy(table_hbm.at[idx_v.at[t * PH + c]],
                                  big[p].at[pl.ds(c * CHUNK, CHUNK)],
                                  gsem[p]).wait()

    def wstart(t, p):
        pltpu.async_copy(big[p], out_hbm.at[pl.ds(base + t * ROWS, ROWS)],
                         osem[p])

    def wwait(t, p):
        pltpu.make_async_copy(big[p],
                              out_hbm.at[pl.ds(base + t * ROWS, ROWS)],
                              osem[p]).wait()

    fire(0, 0)
    drain(0, 0)
    wstart(0, 0)
    fire(1, 1)
    drain(1, 1)
    wstart(1, 1)
    wwait(0, 0)
    fire(2, 0)

    def grp(q, _):
        t = 2 * q
        drain(t, 0)
        wstart(t, 0)
        wwait(t - 1, 1)
        fire(t + 1, 1)
        drain(t + 1, 1)
        wstart(t + 1, 1)
        wwait(t, 0)

        @pl.when(q < NPH // 2 - 1)
        def _():
            fire(t + 2, 0)

        return _

    lax.fori_loop(1, NPH // 2, grp, None)
    wwait(NPH - 1, 1)


@jax.jit
def _embed(ids3, table):
    mesh = plsc.VectorSubcoreMesh(core_axis_name="c", subcore_axis_name="s")
    run = pl.kernel(
        _gather_body,
        out_type=jax.ShapeDtypeStruct((TOTAL, EMBED_DIM), jnp.float32),
        mesh=mesh,
        scratch_types=[
            pltpu.VMEM((K, CHUNK), jnp.int32),
            pltpu.VMEM((ROWS, EMBED_DIM), jnp.float32),
            pltpu.VMEM((ROWS, EMBED_DIM), jnp.float32),
            pltpu.SemaphoreType.DMA,
            pltpu.SemaphoreType.DMA,
            pltpu.SemaphoreType.DMA,
            pltpu.SemaphoreType.DMA,
        ],
        compiler_params=pltpu.CompilerParams(use_tc_tiling_on_sc=False),
    )
    return run(ids3, table)


def kernel(input_ids, embed_tokens_weight):
    ids3 = input_ids.astype(jnp.int32).reshape(NW, K, CHUNK)
    out = _embed(ids3, embed_tokens_weight)
    return out.reshape(BATCH, HIST, EMBED_DIM)


# speedup vs baseline: 1.9862x; 1.0004x over previous
import jax
import jax.numpy as jnp
from jax import lax
from jax.experimental import pallas as pl
from jax.experimental.pallas import tpu as pltpu
from jax.experimental.pallas import tpu_sc as plsc

NC = 2
NS = 16
NW = NC * NS

BATCH = 4096
HIST = 50
EMBED_DIM = 64

TOTAL = BATCH * HIST
PER_W = TOTAL // NW
CHUNK = 128
K = PER_W // CHUNK
PH = 5
ROWS = PH * CHUNK
NPH = K // PH


def _gather_body(ids_hbm, table_hbm, out_hbm, idx_v, big0, big1, gs0, gs1,
                 os0, os1):
    big = (big0, big1)
    gsem = (gs0, gs1)
    osem = (os0, os1)
    wid = lax.axis_index("s") * NC + lax.axis_index("c")
    base = wid * PER_W
    pltpu.sync_copy(ids_hbm.at[wid], idx_v)

    def fire(t, p):
        for c in range(PH):
            pltpu.async_copy(table_hbm.at[idx_v.at[t * PH + c]],
                             big[p].at[pl.ds(c * CHUNK, CHUNK)], gsem[p])

    def drain(t, p):
        for c in range(PH):
            pltpu.make_async_copy(table_hbm.at[idx_v.at[t * PH + c]],
                                  big[p].at[pl.ds(c * CHUNK, CHUNK)],
                                  gsem[p]).wait()

    def wstart(t, p):
        pltpu.async_copy(big[p], out_hbm.at[pl.ds(base + t * ROWS, ROWS)],
                         osem[p])

    def wwait(t, p):
        pltpu.make_async_copy(big[p],
                              out_hbm.at[pl.ds(base + t * ROWS, ROWS)],
                              osem[p]).wait()

    fire(0, 0)
    drain(0, 0)
    wstart(0, 0)
    fire(1, 1)
    drain(1, 1)
    wstart(1, 1)
    wwait(0, 0)
    fire(2, 0)

    def grp(q, _):
        t = 2 * q
        drain(t, 0)
        wstart(t, 0)
        wwait(t - 1, 1)
        fire(t + 1, 1)
        drain(t + 1, 1)
        wstart(t + 1, 1)
        wwait(t, 0)

        @pl.when(q < NPH // 2 - 1)
        def _():
            fire(t + 2, 0)

        return _

    lax.fori_loop(1, NPH // 2, grp, None)
    wwait(NPH - 1, 1)


@jax.jit
def _embed(ids3, table):
    mesh = plsc.VectorSubcoreMesh(core_axis_name="c", subcore_axis_name="s")
    run = pl.kernel(
        _gather_body,
        out_type=jax.ShapeDtypeStruct((TOTAL, EMBED_DIM), jnp.float32),
        mesh=mesh,
        scratch_types=[
            pltpu.VMEM((K, CHUNK), jnp.int32),
            pltpu.VMEM((ROWS, EMBED_DIM), jnp.float32),
            pltpu.VMEM((ROWS, EMBED_DIM), jnp.float32),
            pltpu.SemaphoreType.DMA,
            pltpu.SemaphoreType.DMA,
            pltpu.SemaphoreType.DMA,
            pltpu.SemaphoreType.DMA,
        ],
        compiler_params=pltpu.CompilerParams(use_tc_tiling_on_sc=False,
                                             skip_device_barrier=True),
    )
    return run(ids3, table)


def kernel(input_ids, embed_tokens_weight):
    ids3 = input_ids.astype(jnp.int32).reshape(NW, K, CHUNK)
    out = _embed(ids3, embed_tokens_weight)
    return out.reshape(BATCH, HIST, EMBED_DIM)
